# baseline (speedup 1.0000x reference)
import functools

import jax
import jax.numpy as jnp
from jax import lax
from jax.experimental import pallas as pl
from jax.experimental.pallas import tpu as pltpu
from jax.experimental.pallas import tpu_sc as plsc

_BATCH = 4096
_SEQ = 200
_DIM = 64
_PAD = 128
_B = _BATCH * _SEQ

_CHUNK = 256
_NBUF = 2


@functools.cache
def _build(nw: int, nc: int):
    b_per_w = _B // nw
    n_chunks = b_per_w // _CHUNK
    n_outer = n_chunks // _NBUF
    mesh = plsc.VectorSubcoreMesh(core_axis_name="c", subcore_axis_name="s")

    scratch = (
        [pltpu.VMEM((_CHUNK,), jnp.int32) for _ in range(_NBUF)]
        + [pltpu.VMEM((_CHUNK, _PAD), jnp.float32) for _ in range(_NBUF)]
        + [pltpu.SemaphoreType.DMA for _ in range(3 * _NBUF)]
    )

    @functools.partial(
        pl.kernel,
        out_type=jax.ShapeDtypeStruct((_B, _PAD), jnp.float32),
        mesh=mesh,
        scratch_types=scratch,
        compiler_params=pltpu.CompilerParams(use_tc_tiling_on_sc=False),
    )
    def gather_kernel(x_hbm, table_hbm, out_hbm, *scr):
        idx_v = scr[:_NBUF]
        rows_v = scr[_NBUF:2 * _NBUF]
        isem = scr[2 * _NBUF:3 * _NBUF]
        gsem = scr[3 * _NBUF:4 * _NBUF]
        ssem = scr[4 * _NBUF:5 * _NBUF]

        wid = lax.axis_index("s") * nc + lax.axis_index("c")
        base = wid * b_per_w

        def idx_copy(c, b):
            return pltpu.make_async_copy(
                x_hbm.at[pl.ds(base + c * _CHUNK, _CHUNK)], idx_v[b], isem[b])

        def gather_copy(b):
            return pltpu.make_async_copy(table_hbm.at[idx_v[b]], rows_v[b], gsem[b])

        def store_copy(c, b):
            return pltpu.make_async_copy(
                rows_v[b], out_hbm.at[pl.ds(base + c * _CHUNK, _CHUNK)], ssem[b])

        for b in range(_NBUF):
            idx_copy(b, b).start()

        @pl.loop(0, n_outer)
        def _outer(g):
            for b in range(_NBUF):
                c = g * _NBUF + b
                idx_copy(c, b).wait()

                @pl.when(g >= 1)
                def _():
                    store_copy(c - _NBUF, b).wait()

                gather_copy(b).start()
                gather_copy(b).wait()

                @pl.when(g < n_outer - 1)
                def _():
                    idx_copy(c + _NBUF, b).start()

                store_copy(c, b).start()

        for b in range(_NBUF):
            store_copy((n_outer - 1) * _NBUF + b, b).wait()

    return gather_kernel


_VOC = 1000000
_NCOLS = _VOC // _PAD
_TAIL = _VOC - _NCOLS * _PAD


@functools.cache
def _build_transpose(nw: int, nc: int):
    n_iters = (-(-_NCOLS // nw) + 1) // 2 * 2
    mesh = plsc.VectorSubcoreMesh(core_axis_name="c", subcore_axis_name="s")

    scratch = (
        [pltpu.VMEM((_DIM, _PAD), jnp.float32) for _ in range(2)]
        + [pltpu.VMEM((_PAD, _PAD), jnp.float32) for _ in range(2)]
        + [pltpu.SemaphoreType.DMA for _ in range(4)]
    )

    @functools.partial(
        pl.kernel,
        out_type=jax.ShapeDtypeStruct((_VOC, _PAD), jnp.float32),
        mesh=mesh,
        scratch_types=scratch,
        compiler_params=pltpu.CompilerParams(
            use_tc_tiling_on_sc=True, needs_layout_passes=False),
    )
    def transpose_kernel(tT_hbm, tail_hbm, out_hbm, in0, in1, st0, st1, i0, i1, o0, o1):
        in_v = (in0, in1)
        st_v = (st0, st1)
        isem = (i0, i1)
        osem = (o0, o1)

        wid = lax.axis_index("s") * nc + lax.axis_index("c")

        def col_of(k):
            return jnp.minimum(wid + k * nw, _NCOLS - 1) * _PAD

        def in_copy(k, b):
            return pltpu.make_async_copy(
                tT_hbm.at[:, pl.ds(col_of(k), _PAD)], in_v[b], isem[b])

        def out_copy(k, b):
            return pltpu.make_async_copy(
                st_v[b], out_hbm.at[pl.ds(col_of(k), _PAD)], osem[b])

        def transpose_block(src, dst, nrow):
            @pl.loop(0, nrow)
            def _row(v):
                lvec = lax.iota(jnp.int32, 16) * 0 + v
                for k in range(_DIM // 16):
                    dvec = lax.iota(jnp.int32, 16) + k * 16
                    vals = plsc.load_gather(src, [dvec, lvec])
                    plsc.store_scatter(dst, [lvec, dvec], vals)

        for b in range(2):
            in_copy(b, b).start()

        n_pairs = n_iters // 2

        @pl.loop(0, n_pairs)
        def _main(ko):
            for b in range(2):
                k = ko * 2 + b
                in_copy(k, b).wait()
                transpose_block(in_v[b], st_v[b], _PAD)

                @pl.when(ko >= 1)
                def _():
                    out_copy(k - 2, b).wait()

                out_copy(k, b).start()

                @pl.when(ko < n_pairs - 1)
                def _():
                    in_copy(k + 2, b).start()

        for b in range(2):
            out_copy((n_pairs - 1) * 2 + b, b).wait()

        @pl.when(wid == 0)
        def _tail():
            tin = pltpu.make_async_copy(tail_hbm, in_v[0], isem[0])
            tin.start()
            tin.wait()
            tout = pltpu.make_async_copy(
                in_v[0], out_hbm.at[pl.ds(_NCOLS * _PAD, _TAIL)], osem[0])
            tout.start()
            tout.wait()

    return transpose_kernel


def kernel(x, input_embedding):
    info = plsc.get_sparse_core_info()
    nw = info.num_cores * info.num_subcores
    flat_idx = x.reshape(_B).astype(jnp.int32)
    tail_pad = jnp.pad(input_embedding[_NCOLS * _PAD:], ((0, 0), (0, _PAD - _DIM)))
    table_pad = _build_transpose(nw, info.num_cores)(input_embedding.T, tail_pad)
    out = _build(nw, info.num_cores)(flat_idx, table_pad)
    return out[:, :_DIM].reshape(_BATCH, _SEQ, _DIM)


# speedup vs baseline: 1.1509x; 1.1509x over previous
import functools

import jax
import jax.numpy as jnp
from jax import lax
from jax.experimental import pallas as pl
from jax.experimental.pallas import tpu as pltpu
from jax.experimental.pallas import tpu_sc as plsc

_BATCH = 4096
_SEQ = 200
_DIM = 64
_PAD = 128
_B = _BATCH * _SEQ

_CHUNK = 256
_NBUF = 2


@functools.cache
def _build(nw: int, nc: int):
    b_per_w = _B // nw
    n_chunks = b_per_w // _CHUNK
    n_outer = n_chunks // _NBUF
    mesh = plsc.VectorSubcoreMesh(core_axis_name="c", subcore_axis_name="s")

    scratch = (
        [pltpu.VMEM((_CHUNK,), jnp.int32) for _ in range(_NBUF)]
        + [pltpu.VMEM((_CHUNK, _PAD), jnp.float32) for _ in range(_NBUF)]
        + [pltpu.SemaphoreType.DMA for _ in range(3 * _NBUF)]
    )

    @functools.partial(
        pl.kernel,
        out_type=jax.ShapeDtypeStruct((_B, _PAD), jnp.float32),
        mesh=mesh,
        scratch_types=scratch,
        compiler_params=pltpu.CompilerParams(use_tc_tiling_on_sc=False),
    )
    def gather_kernel(x_hbm, table_hbm, out_hbm, *scr):
        idx_v = scr[:_NBUF]
        rows_v = scr[_NBUF:2 * _NBUF]
        isem = scr[2 * _NBUF:3 * _NBUF]
        gsem = scr[3 * _NBUF:4 * _NBUF]
        ssem = scr[4 * _NBUF:5 * _NBUF]

        wid = lax.axis_index("s") * nc + lax.axis_index("c")
        base = wid * b_per_w

        def idx_copy(c, b):
            return pltpu.make_async_copy(
                x_hbm.at[pl.ds(base + c * _CHUNK, _CHUNK)], idx_v[b], isem[b])

        def gather_copy(b):
            return pltpu.make_async_copy(table_hbm.at[idx_v[b]], rows_v[b], gsem[b])

        def store_copy(c, b):
            return pltpu.make_async_copy(
                rows_v[b], out_hbm.at[pl.ds(base + c * _CHUNK, _CHUNK)], ssem[b])

        for b in range(_NBUF):
            idx_copy(b, b).start()

        @pl.loop(0, n_outer)
        def _outer(g):
            for b in range(_NBUF):
                c = g * _NBUF + b
                idx_copy(c, b).wait()

                @pl.when(g >= 1)
                def _():
                    store_copy(c - _NBUF, b).wait()

                gather_copy(b).start()
                gather_copy(b).wait()

                @pl.when(g < n_outer - 1)
                def _():
                    idx_copy(c + _NBUF, b).start()

                store_copy(c, b).start()

        for b in range(_NBUF):
            store_copy((n_outer - 1) * _NBUF + b, b).wait()

    return gather_kernel


_VOC = 1000000
_NCOLS = _VOC // _PAD
_TAIL = _VOC - _NCOLS * _PAD


@functools.cache
def _build_transpose(nw: int, nc: int):
    n_iters = (-(-_NCOLS // nw) + 1) // 2 * 2
    mesh = plsc.VectorSubcoreMesh(core_axis_name="c", subcore_axis_name="s")

    scratch = (
        [pltpu.VMEM((_DIM, _PAD), jnp.float32) for _ in range(2)]
        + [pltpu.VMEM((_PAD, _PAD), jnp.float32) for _ in range(2)]
        + [pltpu.SemaphoreType.DMA for _ in range(4)]
    )

    @functools.partial(
        pl.kernel,
        out_type=jax.ShapeDtypeStruct((_VOC, _PAD), jnp.float32),
        mesh=mesh,
        scratch_types=scratch,
        compiler_params=pltpu.CompilerParams(
            use_tc_tiling_on_sc=True, needs_layout_passes=False),
    )
    def transpose_kernel(tT_hbm, tail_hbm, out_hbm, in0, in1, st0, st1, i0, i1, o0, o1):
        in_v = (in0, in1)
        st_v = (st0, st1)
        isem = (i0, i1)
        osem = (o0, o1)

        wid = lax.axis_index("s") * nc + lax.axis_index("c")

        def col_of(k):
            return jnp.minimum(wid + k * nw, _NCOLS - 1) * _PAD

        def in_copy(k, b):
            return pltpu.make_async_copy(
                tT_hbm.at[:, pl.ds(col_of(k), _PAD)], in_v[b], isem[b])

        def out_copy(k, b):
            return pltpu.make_async_copy(
                st_v[b], out_hbm.at[pl.ds(col_of(k), _PAD)], osem[b])

        iota = lax.iota(jnp.int32, 16)
        zeros = iota * 0
        vvecs = [iota + g * 16 for g in range(_PAD // 16)]

        def transpose_block(src, dst):
            for d in range(_DIM):
                dvec = zeros + d
                for g in range(_PAD // 16):
                    vals = src[d, pl.ds(g * 16, 16)]
                    plsc.store_scatter(dst, [vvecs[g], dvec], vals)

        for b in range(2):
            in_copy(b, b).start()

        n_pairs = n_iters // 2

        @pl.loop(0, n_pairs)
        def _main(ko):
            for b in range(2):
                k = ko * 2 + b
                in_copy(k, b).wait()
                transpose_block(in_v[b], st_v[b])

                @pl.when(ko >= 1)
                def _():
                    out_copy(k - 2, b).wait()

                out_copy(k, b).start()

                @pl.when(ko < n_pairs - 1)
                def _():
                    in_copy(k + 2, b).start()

        for b in range(2):
            out_copy((n_pairs - 1) * 2 + b, b).wait()

        @pl.when(wid == 0)
        def _tail():
            tin = pltpu.make_async_copy(tail_hbm, in_v[0], isem[0])
            tin.start()
            tin.wait()
            tout = pltpu.make_async_copy(
                in_v[0], out_hbm.at[pl.ds(_NCOLS * _PAD, _TAIL)], osem[0])
            tout.start()
            tout.wait()

    return transpose_kernel


def kernel(x, input_embedding):
    info = plsc.get_sparse_core_info()
    nw = info.num_cores * info.num_subcores
    flat_idx = x.reshape(_B).astype(jnp.int32)
    tail_pad = jnp.pad(input_embedding[_NCOLS * _PAD:], ((0, 0), (0, _PAD - _DIM)))
    table_pad = _build_transpose(nw, info.num_cores)(input_embedding.T, tail_pad)
    out = _build(nw, info.num_cores)(flat_idx, table_pad)
    return out[:, :_DIM].reshape(_BATCH, _SEQ, _DIM)
